# Initial kernel scaffold; baseline (speedup 1.0000x reference)
import jax
import jax.numpy as jnp
from jax.experimental import pallas as pl


def kernel(target, source, target_feats, source_feats, W0, gamma0, beta0):
    raise NotImplementedError("write your pallas kernel here")


# speedup vs baseline: 3.5515x; 3.5515x over previous
import jax
import jax.numpy as jnp
from jax import lax
from jax.experimental import pallas as pl
from jax.experimental.pallas import tpu as pltpu
from jax.experimental.pallas import tpu_sc as plsc

B, N, M, C = 8, 4096, 1024, 64
TN = 512
TC_ = 512
TD = 512
BIG = 3.4e38


def _knn_body(t_ref, st_ref, idx_ref, w_ref):
    T = t_ref[0]
    St = st_ref[0]
    dot = lax.dot_general(T, St, (((1,), (0,)), ((), ())),
                          preferred_element_type=jnp.float32)
    t2 = jnp.sum(T * T, axis=1, keepdims=True)
    s2 = jnp.sum(St * St, axis=0, keepdims=True)
    d = t2 - 2.0 * dot + s2
    iota = lax.broadcasted_iota(jnp.int32, (TN, M), 1)
    ivs, rvs = [], []
    for _ in range(3):
        mv = jnp.min(d, axis=1, keepdims=True)
        cand = jnp.where(d == mv, iota, M)
        iv = jnp.min(cand, axis=1, keepdims=True)
        d = jnp.where(cand == iv, BIG, d)
        dist = jnp.sqrt(jnp.maximum(mv, 1e-12))
        ivs.append(iv)
        rvs.append(1.0 / (dist + 1e-8))
    rsum = rvs[0] + rvs[1] + rvs[2]
    idx_ref[0] = jnp.concatenate(ivs, axis=1)
    w_ref[0] = jnp.concatenate([r / rsum for r in rvs], axis=1)


def _mlp_body(it_ref, tf_ref, w0_ref, y_ref, st_ref):
    step = pl.program_id(0) * (N // TC_) + pl.program_id(1)
    it = it_ref[0]
    tf = tf_ref[0]
    W0 = w0_ref[...]
    dn = (((1,), (0,)), ((), ()))
    y = (lax.dot_general(W0[:, :C], it, dn, preferred_element_type=jnp.float32)
         + lax.dot_general(W0[:, C:], tf, dn, preferred_element_type=jnp.float32))
    y_ref[0] = y
    s1 = jnp.sum(y, axis=1, keepdims=True)
    s2 = jnp.sum(y * y, axis=1, keepdims=True)
    s = jnp.concatenate([s1, s2], axis=1)

    @pl.when(step == 0)
    def _():
        st_ref[...] = s

    @pl.when(step != 0)
    def _():
        st_ref[...] += s


def _norm_body(y_ref, ss_ref, o_ref):
    y = y_ref[0]
    sc = ss_ref[:, 0:1]
    sh = ss_ref[:, 1:2]
    o_ref[0] = jnp.maximum(y * sc + sh, 0.0)


def kernel(target, source, target_feats, source_feats, W0, gamma0, beta0):
    sourceT = jnp.swapaxes(source, 1, 2)

    idx, w = pl.pallas_call(
        _knn_body,
        grid=(B, N // TN),
        in_specs=[
            pl.BlockSpec((1, TN, 3), lambda b, j: (b, j, 0)),
            pl.BlockSpec((1, 3, M), lambda b, j: (b, 0, 0)),
        ],
        out_specs=[
            pl.BlockSpec((1, TN, 3), lambda b, j: (b, j, 0)),
            pl.BlockSpec((1, TN, 3), lambda b, j: (b, j, 0)),
        ],
        out_shape=[
            jax.ShapeDtypeStruct((B, N, 3), jnp.int32),
            jax.ShapeDtypeStruct((B, N, 3), jnp.float32),
        ],
    )(target, sourceT)

    gathered = jax.vmap(lambda sf, ix: sf[:, ix])(source_feats, idx)
    interp = jnp.sum(gathered * w[:, None, :, :], axis=-1)

    y, st = pl.pallas_call(
        _mlp_body,
        grid=(B, N // TC_),
        in_specs=[
            pl.BlockSpec((1, C, TC_), lambda b, j: (b, 0, j)),
            pl.BlockSpec((1, C, TC_), lambda b, j: (b, 0, j)),
            pl.BlockSpec((C, 2 * C), lambda b, j: (0, 0)),
        ],
        out_specs=[
            pl.BlockSpec((1, C, TC_), lambda b, j: (b, 0, j)),
            pl.BlockSpec((C, 2), lambda b, j: (0, 0)),
        ],
        out_shape=[
            jax.ShapeDtypeStruct((B, C, N), jnp.float32),
            jax.ShapeDtypeStruct((C, 2), jnp.float32),
        ],
    )(interp, target_feats, W0)

    cnt = jnp.float32(B * N)
    mean = st[:, 0] / cnt
    var = st[:, 1] / cnt - mean * mean
    scale = gamma0 / jnp.sqrt(var + 1e-5)
    shift = beta0 - mean * scale
    scsh = jnp.stack([scale, shift], axis=1)

    out = pl.pallas_call(
        _norm_body,
        grid=(B, N // TD),
        in_specs=[
            pl.BlockSpec((1, C, TD), lambda b, j: (b, 0, j)),
            pl.BlockSpec((C, 2), lambda b, j: (0, 0)),
        ],
        out_specs=pl.BlockSpec((1, C, TD), lambda b, j: (b, 0, j)),
        out_shape=jax.ShapeDtypeStruct((B, C, N), jnp.float32),
    )(y, scsh)
    return out


# speedup vs baseline: 18.8624x; 5.3112x over previous
import jax
import jax.numpy as jnp
from jax import lax
from jax.experimental import pallas as pl
from jax.experimental.pallas import tpu as pltpu
from jax.experimental.pallas import tpu_sc as plsc

B, N, M, C = 8, 4096, 1024, 64
TN = 512
TC_ = 512
TD = 512
BIG = 3.4e38


def _knn_body(t_ref, st_ref, idx_ref, w_ref):
    T = t_ref[0]
    St = st_ref[0]
    dot = lax.dot_general(T, St, (((1,), (0,)), ((), ())),
                          preferred_element_type=jnp.float32)
    t2 = jnp.sum(T * T, axis=1, keepdims=True)
    s2 = jnp.sum(St * St, axis=0, keepdims=True)
    d = t2 - 2.0 * dot + s2
    iota = lax.broadcasted_iota(jnp.int32, (TN, M), 1)
    ivs, rvs = [], []
    for _ in range(3):
        mv = jnp.min(d, axis=1, keepdims=True)
        cand = jnp.where(d == mv, iota, M)
        iv = jnp.min(cand, axis=1, keepdims=True)
        d = jnp.where(cand == iv, BIG, d)
        dist = jnp.sqrt(jnp.maximum(mv, 1e-12))
        ivs.append(iv)
        rvs.append(1.0 / (dist + 1e-8))
    rsum = rvs[0] + rvs[1] + rvs[2]
    idx_ref[0] = jnp.concatenate(ivs, axis=1)
    w_ref[0] = jnp.concatenate([r / rsum for r in rvs], axis=1)


def _mlp_body(it_ref, tf_ref, w0_ref, y_ref, st_ref):
    step = pl.program_id(0) * (N // TC_) + pl.program_id(1)
    it = it_ref[0]
    tf = tf_ref[0]
    W0 = w0_ref[...]
    dn = (((1,), (0,)), ((), ()))
    y = (lax.dot_general(W0[:, :C], it, dn, preferred_element_type=jnp.float32)
         + lax.dot_general(W0[:, C:], tf, dn, preferred_element_type=jnp.float32))
    y_ref[0] = y
    s1 = jnp.sum(y, axis=1, keepdims=True)
    s2 = jnp.sum(y * y, axis=1, keepdims=True)
    s = jnp.concatenate([s1, s2], axis=1)

    @pl.when(step == 0)
    def _():
        st_ref[...] = s

    @pl.when(step != 0)
    def _():
        st_ref[...] += s


_NW = 32
_WPB = _NW // B
_NPW = N // _WPB
_CHUNK = 512


def _interp_sc_body(sf_hbm, idx_hbm, w_hbm, out_hbm, table_v, idx_v, w_v,
                    out_v):
    wid = lax.axis_index("s") * 2 + lax.axis_index("c")
    b = wid // _WPB
    base = (wid % _WPB) * _NPW
    pltpu.sync_copy(sf_hbm.at[b], table_v)
    pltpu.sync_copy(idx_hbm.at[b, :, pl.ds(base, _NPW)], idx_v)
    pltpu.sync_copy(w_hbm.at[b, :, pl.ds(base, _NPW)], w_v)
    for chunk in range(_NPW // _CHUNK):
        coff = chunk * _CHUNK

        def body(g, carry):
            s = coff + g * 16
            i0 = idx_v[0, pl.ds(s, 16)]
            i1 = idx_v[1, pl.ds(s, 16)]
            i2 = idx_v[2, pl.ds(s, 16)]
            w0 = w_v[0, pl.ds(s, 16)]
            w1 = w_v[1, pl.ds(s, 16)]
            w2 = w_v[2, pl.ds(s, 16)]
            for c in range(C):
                cs = jnp.full((16,), c, jnp.int32)
                v = (w0 * plsc.load_gather(table_v, [cs, i0])
                     + w1 * plsc.load_gather(table_v, [cs, i1])
                     + w2 * plsc.load_gather(table_v, [cs, i2]))
                out_v[c, pl.ds(g * 16, 16)] = v
            return carry

        lax.fori_loop(0, _CHUNK // 16, body, 0)
        pltpu.sync_copy(out_v, out_hbm.at[b, :, pl.ds(base + coff, _CHUNK)])


def _interp_sc(source_feats, idx_t, w_t):
    mesh = plsc.VectorSubcoreMesh(core_axis_name="c", subcore_axis_name="s")
    return pl.kernel(
        _interp_sc_body,
        out_type=jax.ShapeDtypeStruct((B, C, N), jnp.float32),
        mesh=mesh,
        scratch_types=[
            pltpu.VMEM((C, M), jnp.float32),
            pltpu.VMEM((3, _NPW), jnp.int32),
            pltpu.VMEM((3, _NPW), jnp.float32),
            pltpu.VMEM((C, _CHUNK), jnp.float32),
        ],
        compiler_params=pltpu.CompilerParams(use_tc_tiling_on_sc=False,
                                             needs_layout_passes=False),
    )(source_feats, idx_t, w_t)


def _norm_body(y_ref, ss_ref, o_ref):
    y = y_ref[0]
    sc = ss_ref[:, 0:1]
    sh = ss_ref[:, 1:2]
    o_ref[0] = jnp.maximum(y * sc + sh, 0.0)


def kernel(target, source, target_feats, source_feats, W0, gamma0, beta0):
    sourceT = jnp.swapaxes(source, 1, 2)

    idx, w = pl.pallas_call(
        _knn_body,
        grid=(B, N // TN),
        in_specs=[
            pl.BlockSpec((1, TN, 3), lambda b, j: (b, j, 0)),
            pl.BlockSpec((1, 3, M), lambda b, j: (b, 0, 0)),
        ],
        out_specs=[
            pl.BlockSpec((1, TN, 3), lambda b, j: (b, j, 0)),
            pl.BlockSpec((1, TN, 3), lambda b, j: (b, j, 0)),
        ],
        out_shape=[
            jax.ShapeDtypeStruct((B, N, 3), jnp.int32),
            jax.ShapeDtypeStruct((B, N, 3), jnp.float32),
        ],
    )(target, sourceT)

    idx_t = jnp.swapaxes(idx, 1, 2)
    w_t = jnp.swapaxes(w, 1, 2)
    interp = _interp_sc(source_feats, idx_t, w_t)

    y, st = pl.pallas_call(
        _mlp_body,
        grid=(B, N // TC_),
        in_specs=[
            pl.BlockSpec((1, C, TC_), lambda b, j: (b, 0, j)),
            pl.BlockSpec((1, C, TC_), lambda b, j: (b, 0, j)),
            pl.BlockSpec((C, 2 * C), lambda b, j: (0, 0)),
        ],
        out_specs=[
            pl.BlockSpec((1, C, TC_), lambda b, j: (b, 0, j)),
            pl.BlockSpec((C, 2), lambda b, j: (0, 0)),
        ],
        out_shape=[
            jax.ShapeDtypeStruct((B, C, N), jnp.float32),
            jax.ShapeDtypeStruct((C, 2), jnp.float32),
        ],
    )(interp, target_feats, W0)

    cnt = jnp.float32(B * N)
    mean = st[:, 0] / cnt
    var = st[:, 1] / cnt - mean * mean
    scale = gamma0 / jnp.sqrt(var + 1e-5)
    shift = beta0 - mean * scale
    scsh = jnp.stack([scale, shift], axis=1)

    out = pl.pallas_call(
        _norm_body,
        grid=(B, N // TD),
        in_specs=[
            pl.BlockSpec((1, C, TD), lambda b, j: (b, 0, j)),
            pl.BlockSpec((C, 2), lambda b, j: (0, 0)),
        ],
        out_specs=pl.BlockSpec((1, C, TD), lambda b, j: (b, 0, j)),
        out_shape=jax.ShapeDtypeStruct((B, C, N), jnp.float32),
    )(y, scsh)
    return out
